# Initial kernel scaffold; baseline (speedup 1.0000x reference)
import jax
import jax.numpy as jnp
from jax.experimental import pallas as pl


def kernel(x, edge_index, W, b):
    raise NotImplementedError("write your pallas kernel here")


# speedup vs baseline: 23.5124x; 23.5124x over previous
import jax
import jax.numpy as jnp
from jax import lax
from jax.experimental import pallas as pl
from jax.experimental.pallas import tpu as pltpu
from jax.experimental.pallas import tpu_sc as plsc

N = 10000
D = 128
HD = D // 2
E = 320000

NPAD = 10240
NC, NS = 2, 16
NW = NC * NS
CK = 128
CHD = E // (NW * CK)
EPAD = NW * CK * ((E + NW * CK - 1) // (NW * CK))
CH = EPAD // (NS * CK)
RPS = NPAD // NS

_mesh = plsc.VectorSubcoreMesh(core_axis_name="c", subcore_axis_name="s")


def _deg_body(dst_hbm, zeros_hbm, ones_hbm, out_hbm, dst_v, ones_v, deg_sh):
    cid = lax.axis_index("c")
    sid = lax.axis_index("s")
    wid = cid * NS + sid
    pltpu.sync_copy(zeros_hbm.at[pl.ds(sid * RPS, RPS)],
                    deg_sh.at[pl.ds(sid * RPS, RPS)])
    pltpu.sync_copy(ones_hbm, ones_v)
    pltpu.sync_copy(dst_hbm.at[wid], dst_v)
    plsc.subcore_barrier()

    @pl.loop(0, EPAD // (NW * CK))
    def _(j):
        pltpu.sync_copy(ones_v, deg_sh.at[dst_v.at[j]], add=True)

    plsc.subcore_barrier()
    pltpu.sync_copy(deg_sh.at[pl.ds(sid * RPS, RPS)],
                    out_hbm.at[cid, pl.ds(sid * RPS, RPS)])


_sc_deg = pl.kernel(
    _deg_body,
    out_type=jax.ShapeDtypeStruct((NC, NPAD, 16), jnp.float32),
    mesh=_mesh,
    scratch_types=[
        pltpu.VMEM((EPAD // (NW * CK), CK), jnp.int32),
        pltpu.VMEM((CK, 16), jnp.float32),
        pltpu.VMEM_SHARED((NPAD, 16), jnp.float32),
    ],
)


def _agg_pipeline(g_hbm, src_v, dst_v, buf0, buf1, s_sh, sem0, sem1):
    pltpu.async_copy(g_hbm.at[src_v.at[0]], buf0, sem0)

    @pl.loop(0, CH // 2)
    def _(it):
        j0 = it * 2
        pltpu.async_copy(g_hbm.at[src_v.at[j0 + 1]], buf1, sem1)
        pltpu.make_async_copy(g_hbm.at[src_v.at[j0]], buf0, sem0).wait()
        pltpu.sync_copy(buf0, s_sh.at[dst_v.at[j0]], add=True)

        @pl.when(it < CH // 2 - 1)
        def _():
            pltpu.async_copy(g_hbm.at[src_v.at[j0 + 2]], buf0, sem0)

        pltpu.make_async_copy(g_hbm.at[src_v.at[j0 + 1]], buf1, sem1).wait()
        pltpu.sync_copy(buf1, s_sh.at[dst_v.at[j0 + 1]], add=True)


def _agg_body(gl_hbm, gr_hbm, src_hbm, dst_hbm, zeros_hbm, out_hbm,
              src_v, dst_v, buf0, buf1, s_sh, sem0, sem1):
    cid = lax.axis_index("c")
    sid = lax.axis_index("s")
    pltpu.sync_copy(src_hbm.at[sid], src_v)
    pltpu.sync_copy(dst_hbm.at[sid], dst_v)
    pltpu.sync_copy(zeros_hbm.at[pl.ds(sid * RPS, RPS)],
                    s_sh.at[pl.ds(sid * RPS, RPS)])
    plsc.subcore_barrier()

    @pl.when(cid == 0)
    def _():
        _agg_pipeline(gl_hbm, src_v, dst_v, buf0, buf1, s_sh, sem0, sem1)

    @pl.when(cid == 1)
    def _():
        _agg_pipeline(gr_hbm, src_v, dst_v, buf0, buf1, s_sh, sem0, sem1)

    plsc.subcore_barrier()
    pltpu.sync_copy(s_sh.at[pl.ds(sid * RPS, RPS)],
                    out_hbm.at[cid, pl.ds(sid * RPS, RPS)])


_sc_agg = pl.kernel(
    _agg_body,
    out_type=jax.ShapeDtypeStruct((NC, NPAD, HD), jnp.float32),
    mesh=_mesh,
    scratch_types=[
        pltpu.VMEM((CH, CK), jnp.int32),
        pltpu.VMEM((CH, CK), jnp.int32),
        pltpu.VMEM((CK, HD), jnp.float32),
        pltpu.VMEM((CK, HD), jnp.float32),
        pltpu.VMEM_SHARED((NPAD, HD), jnp.float32),
        pltpu.SemaphoreType.DMA,
        pltpu.SemaphoreType.DMA,
    ],
    compiler_params=pltpu.CompilerParams(use_tc_tiling_on_sc=False),
)


def _g_body(x_ref, w_ref, d0_ref, d1_ref, gl_ref, gr_ref):
    deg = d0_ref[:, 0:1] + d1_ref[:, 0:1] + 1.0
    h = jnp.dot(x_ref[...], w_ref[...],
                preferred_element_type=jnp.float32,
                precision=lax.Precision.HIGHEST)
    g = h * lax.rsqrt(deg)
    gl_ref[...] = g[:, :HD]
    gr_ref[...] = g[:, HD:]


_BLK1 = 1024


def _tc_g(x_pad, W, d0, d1):
    return pl.pallas_call(
        _g_body,
        grid=(NPAD // _BLK1,),
        in_specs=[
            pl.BlockSpec((_BLK1, D), lambda i: (i, 0)),
            pl.BlockSpec((D, D), lambda i: (0, 0)),
            pl.BlockSpec((_BLK1, 16), lambda i: (i, 0)),
            pl.BlockSpec((_BLK1, 16), lambda i: (i, 0)),
        ],
        out_specs=[
            pl.BlockSpec((_BLK1, HD), lambda i: (i, 0)),
            pl.BlockSpec((_BLK1, HD), lambda i: (i, 0)),
        ],
        out_shape=[
            jax.ShapeDtypeStruct((NPAD, HD), jnp.float32),
            jax.ShapeDtypeStruct((NPAD, HD), jnp.float32),
        ],
    )(x_pad, W, d0, d1)


def _out_body(s0_ref, s1_ref, gl_ref, gr_ref, d0_ref, d1_ref, b_ref, o_ref):
    deg = d0_ref[:, 0:1] + d1_ref[:, 0:1] + 1.0
    r = lax.rsqrt(deg)
    s = jnp.concatenate([s0_ref[...] + gl_ref[...],
                         s1_ref[...] + gr_ref[...]], axis=1)
    o_ref[...] = jnp.tanh(s * r + b_ref[...])


_BLK2 = 1000


def _tc_out(s0, s1, gl, gr, d0, d1, b2):
    return pl.pallas_call(
        _out_body,
        grid=(N // _BLK2,),
        in_specs=[
            pl.BlockSpec((_BLK2, HD), lambda i: (i, 0)),
            pl.BlockSpec((_BLK2, HD), lambda i: (i, 0)),
            pl.BlockSpec((_BLK2, HD), lambda i: (i, 0)),
            pl.BlockSpec((_BLK2, HD), lambda i: (i, 0)),
            pl.BlockSpec((_BLK2, 16), lambda i: (i, 0)),
            pl.BlockSpec((_BLK2, 16), lambda i: (i, 0)),
            pl.BlockSpec((1, D), lambda i: (0, 0)),
        ],
        out_specs=pl.BlockSpec((_BLK2, D), lambda i: (i, 0)),
        out_shape=jax.ShapeDtypeStruct((N, D), jnp.float32),
    )(s0, s1, gl, gr, d0, d1, b2)


def kernel(x, edge_index, W, b):
    src = edge_index[0].astype(jnp.int32)
    dst = edge_index[1].astype(jnp.int32)
    pad = jnp.full((EPAD - E,), N, dtype=jnp.int32)
    src3 = jnp.concatenate([src, pad]).reshape(NS, CH, CK)
    dst3 = jnp.concatenate([dst, pad]).reshape(NS, CH, CK)
    dst3w = dst3.reshape(NW, EPAD // (NW * CK), CK)
    x_pad = jnp.pad(x, ((0, NPAD - N), (0, 0)))

    zeros16 = jnp.zeros((NPAD, 16), jnp.float32)
    ones16 = jnp.ones((CK, 16), jnp.float32)
    zeros64 = jnp.zeros((NPAD, HD), jnp.float32)

    degp = _sc_deg(dst3w, zeros16, ones16)
    gl, gr = _tc_g(x_pad, W, degp[0], degp[1])
    S = _sc_agg(gl, gr, src3, dst3, zeros64)
    return _tc_out(S[0], S[1], gl, gr, degp[0], degp[1],
                   b.reshape(1, D).astype(jnp.float32))


# speedup vs baseline: 25.5036x; 1.0847x over previous
import jax
import jax.numpy as jnp
from jax import lax
from jax.experimental import pallas as pl
from jax.experimental.pallas import tpu as pltpu
from jax.experimental.pallas import tpu_sc as plsc

N = 10000
D = 128
HD = D // 2
E = 320000

NPAD = 10240
NC, NS = 2, 16
NW = NC * NS
CK = 128
CHD = E // (NW * CK)
EPAD = NW * CK * ((E + NW * CK - 1) // (NW * CK))
CH = EPAD // (NS * CK)
RPS = NPAD // NS

_mesh = plsc.VectorSubcoreMesh(core_axis_name="c", subcore_axis_name="s")


def _deg_body(dst_hbm, zeros_hbm, ones_hbm, out_hbm, dst_v, ones_v, deg_sh):
    cid = lax.axis_index("c")
    sid = lax.axis_index("s")
    wid = cid * NS + sid
    pltpu.sync_copy(zeros_hbm.at[pl.ds(sid * RPS, RPS)],
                    deg_sh.at[pl.ds(sid * RPS, RPS)])
    pltpu.sync_copy(ones_hbm, ones_v)
    pltpu.sync_copy(dst_hbm.at[wid], dst_v)
    plsc.subcore_barrier()

    @pl.loop(0, EPAD // (NW * CK))
    def _(j):
        pltpu.sync_copy(ones_v, deg_sh.at[dst_v.at[j]], add=True)

    plsc.subcore_barrier()
    pltpu.sync_copy(deg_sh.at[pl.ds(sid * RPS, RPS)],
                    out_hbm.at[cid, pl.ds(sid * RPS, RPS)])


_sc_deg = pl.kernel(
    _deg_body,
    out_type=jax.ShapeDtypeStruct((NC, NPAD, 16), jnp.float32),
    mesh=_mesh,
    scratch_types=[
        pltpu.VMEM((EPAD // (NW * CK), CK), jnp.int32),
        pltpu.VMEM((CK, 16), jnp.float32),
        pltpu.VMEM_SHARED((NPAD, 16), jnp.float32),
    ],
)


NB = 4
OFF = 3


def _agg_pipeline(g_hbm, src_v, dst_v, bufs, s_sh, sems):
    def g_issue(j, b):
        pltpu.async_copy(g_hbm.at[src_v.at[j]], bufs[b], sems[b])

    def g_wait(j, b):
        pltpu.make_async_copy(g_hbm.at[src_v.at[j]], bufs[b],
                              sems[b]).wait()

    def step(j, b, gissue):
        g_wait(j, b)
        if gissue:
            g_issue(j + OFF, (b + OFF) % NB)
        pltpu.sync_copy(bufs[b], s_sh.at[dst_v.at[j]], add=True)

    for j in range(OFF):
        g_issue(j, j)

    @pl.loop(0, CH // NB - 1)
    def _(it):
        j0 = it * NB
        for b in range(NB):
            step(j0 + b, b, True)

    for b in range(NB):
        j = CH - NB + b
        step(j, b, j + OFF < CH)


def _agg_body(gl_hbm, gr_hbm, src_hbm, dst_hbm, zeros_hbm, out_hbm,
              src_v, dst_v, b0, b1, b2, b3,
              s_sh, sem0, sem1, sem2, sem3):
    cid = lax.axis_index("c")
    sid = lax.axis_index("s")
    bufs = (b0, b1, b2, b3)
    sems = (sem0, sem1, sem2, sem3)
    pltpu.sync_copy(src_hbm.at[sid], src_v)
    pltpu.sync_copy(dst_hbm.at[sid], dst_v)
    pltpu.sync_copy(zeros_hbm.at[pl.ds(sid * RPS, RPS)],
                    s_sh.at[pl.ds(sid * RPS, RPS)])
    plsc.subcore_barrier()

    @pl.when(cid == 0)
    def _():
        _agg_pipeline(gl_hbm, src_v, dst_v, bufs, s_sh, sems)

    @pl.when(cid == 1)
    def _():
        _agg_pipeline(gr_hbm, src_v, dst_v, bufs, s_sh, sems)

    plsc.subcore_barrier()
    pltpu.sync_copy(s_sh.at[pl.ds(sid * RPS, RPS)],
                    out_hbm.at[cid, pl.ds(sid * RPS, RPS)])


_sc_agg = pl.kernel(
    _agg_body,
    out_type=jax.ShapeDtypeStruct((NC, NPAD, HD), jnp.float32),
    mesh=_mesh,
    scratch_types=[
        pltpu.VMEM((CH, CK), jnp.int32),
        pltpu.VMEM((CH, CK), jnp.int32),
    ] + [pltpu.VMEM((CK, HD), jnp.float32) for _ in range(NB)] + [
        pltpu.VMEM_SHARED((NPAD, HD), jnp.float32),
    ] + [pltpu.SemaphoreType.DMA for _ in range(NB)],
    compiler_params=pltpu.CompilerParams(use_tc_tiling_on_sc=False),
)


def _h_body(x_ref, w_ref, h_ref):
    h_ref[...] = jnp.dot(x_ref[...], w_ref[...],
                         preferred_element_type=jnp.float32,
                         precision=lax.Precision.HIGHEST)


_BLK1 = 1024


def _tc_h(x_pad, W):
    return pl.pallas_call(
        _h_body,
        grid=(NPAD // _BLK1,),
        in_specs=[
            pl.BlockSpec((_BLK1, D), lambda i: (i, 0)),
            pl.BlockSpec((D, D), lambda i: (0, 0)),
        ],
        out_specs=pl.BlockSpec((_BLK1, D), lambda i: (i, 0)),
        out_shape=jax.ShapeDtypeStruct((NPAD, D), jnp.float32),
    )(x_pad, W)


def _g_body(h_ref, d0_ref, d1_ref, gl_ref, gr_ref):
    deg = d0_ref[:, 0:1] + d1_ref[:, 0:1] + 1.0
    g = h_ref[...] * lax.rsqrt(deg)
    gl_ref[...] = g[:, :HD]
    gr_ref[...] = g[:, HD:]


def _tc_g(h, d0, d1):
    return pl.pallas_call(
        _g_body,
        grid=(NPAD // _BLK1,),
        in_specs=[
            pl.BlockSpec((_BLK1, D), lambda i: (i, 0)),
            pl.BlockSpec((_BLK1, 16), lambda i: (i, 0)),
            pl.BlockSpec((_BLK1, 16), lambda i: (i, 0)),
        ],
        out_specs=[
            pl.BlockSpec((_BLK1, HD), lambda i: (i, 0)),
            pl.BlockSpec((_BLK1, HD), lambda i: (i, 0)),
        ],
        out_shape=[
            jax.ShapeDtypeStruct((NPAD, HD), jnp.float32),
            jax.ShapeDtypeStruct((NPAD, HD), jnp.float32),
        ],
    )(h, d0, d1)


def _out_body(s0_ref, s1_ref, gl_ref, gr_ref, d0_ref, d1_ref, b_ref, o_ref):
    deg = d0_ref[:, 0:1] + d1_ref[:, 0:1] + 1.0
    r = lax.rsqrt(deg)
    s = jnp.concatenate([s0_ref[...] + gl_ref[...],
                         s1_ref[...] + gr_ref[...]], axis=1)
    o_ref[...] = jnp.tanh(s * r + b_ref[...])


_BLK2 = 1000


def _tc_out(s0, s1, gl, gr, d0, d1, b2):
    return pl.pallas_call(
        _out_body,
        grid=(N // _BLK2,),
        in_specs=[
            pl.BlockSpec((_BLK2, HD), lambda i: (i, 0)),
            pl.BlockSpec((_BLK2, HD), lambda i: (i, 0)),
            pl.BlockSpec((_BLK2, HD), lambda i: (i, 0)),
            pl.BlockSpec((_BLK2, HD), lambda i: (i, 0)),
            pl.BlockSpec((_BLK2, 16), lambda i: (i, 0)),
            pl.BlockSpec((_BLK2, 16), lambda i: (i, 0)),
            pl.BlockSpec((1, D), lambda i: (0, 0)),
        ],
        out_specs=pl.BlockSpec((_BLK2, D), lambda i: (i, 0)),
        out_shape=jax.ShapeDtypeStruct((N, D), jnp.float32),
    )(s0, s1, gl, gr, d0, d1, b2)


def kernel(x, edge_index, W, b):
    src = edge_index[0].astype(jnp.int32)
    dst = edge_index[1].astype(jnp.int32)
    pad = jnp.full((EPAD - E,), N, dtype=jnp.int32)
    src3 = jnp.concatenate([src, pad]).reshape(NS, CH, CK)
    dst3 = jnp.concatenate([dst, pad]).reshape(NS, CH, CK)
    dst3w = dst3.reshape(NW, EPAD // (NW * CK), CK)
    x_pad = jnp.pad(x, ((0, NPAD - N), (0, 0)))

    zeros16 = jnp.zeros((NPAD, 16), jnp.float32)
    ones16 = jnp.ones((CK, 16), jnp.float32)
    zeros64 = jnp.zeros((NPAD, HD), jnp.float32)

    degp = _sc_deg(dst3w, zeros16, ones16)
    h = _tc_h(x_pad, W)
    gl, gr = _tc_g(h, degp[0], degp[1])
    S = _sc_agg(gl, gr, src3, dst3, zeros64)
    return _tc_out(S[0], S[1], gl, gr, degp[0], degp[1],
                   b.reshape(1, D).astype(jnp.float32))
